# Initial kernel scaffold; baseline (speedup 1.0000x reference)
import jax
import jax.numpy as jnp
from jax.experimental import pallas as pl


def kernel(multi_hot, table):
    raise NotImplementedError("write your pallas kernel here")


# speedup vs baseline: 4.0756x; 4.0756x over previous
import functools

import jax
import jax.numpy as jnp
from jax import lax
from jax.experimental import pallas as pl
from jax.experimental.pallas import tpu as pltpu
from jax.experimental.pallas import tpu_sc as plsc

EMB = 64
NC, NS = 2, 16
NW = NC * NS
CHUNK = 128


@functools.cache
def _make_gather(B: int):
    bpw = B // NW
    nchunk = bpw // CHUNK
    mesh = plsc.VectorSubcoreMesh(core_axis_name="c", subcore_axis_name="s")

    @functools.partial(
        pl.kernel,
        out_type=jax.ShapeDtypeStruct((B, EMB), jnp.float32),
        mesh=mesh,
        compiler_params=pltpu.CompilerParams(use_tc_tiling_on_sc=False),
        scratch_types=[
            pltpu.VMEM((nchunk, CHUNK), jnp.int32),
            pltpu.VMEM((CHUNK, EMB), jnp.float32),
            pltpu.SemaphoreType.DMA,
        ],
    )
    def gather_kernel(idx_hbm, table_hbm, out_hbm, idx_v, rows_v, gsem):
        wid = lax.axis_index("s") * NC + lax.axis_index("c")
        base = wid * bpw
        pltpu.sync_copy(idx_hbm.at[wid], idx_v)

        def body(j, _):
            pltpu.async_copy(table_hbm.at[idx_v.at[j]], rows_v, gsem).wait()
            pltpu.sync_copy(
                rows_v, out_hbm.at[pl.ds(base + j * CHUNK, CHUNK)])
            return ()

        lax.fori_loop(0, nchunk, body, (), unroll=False)

    return gather_kernel


def kernel(multi_hot, table):
    rows, cols = multi_hot.shape
    B = rows * cols
    idx3 = multi_hot.astype(jnp.int32).reshape(NW, B // (NW * CHUNK), CHUNK)
    out = _make_gather(B)(idx3, table)
    return out.reshape(rows, cols, EMB)


# speedup vs baseline: 4.6128x; 1.1318x over previous
import functools

import jax
import jax.numpy as jnp
from jax import lax
from jax.experimental import pallas as pl
from jax.experimental.pallas import tpu as pltpu
from jax.experimental.pallas import tpu_sc as plsc

EMB = 64
NC, NS = 2, 16
NW = NC * NS
CHUNK = 128
K = 5
SUPER = CHUNK * K


@functools.cache
def _make_gather(B: int):
    bpw = B // NW
    nchunk = bpw // CHUNK
    nsuper = nchunk // K
    mesh = plsc.VectorSubcoreMesh(core_axis_name="c", subcore_axis_name="s")

    @functools.partial(
        pl.kernel,
        out_type=jax.ShapeDtypeStruct((B, EMB), jnp.float32),
        mesh=mesh,
        compiler_params=pltpu.CompilerParams(use_tc_tiling_on_sc=False),
        scratch_types=[
            pltpu.VMEM((nchunk, CHUNK), jnp.int32),
            pltpu.VMEM((SUPER, EMB), jnp.float32),
            pltpu.VMEM((SUPER, EMB), jnp.float32),
            pltpu.SemaphoreType.DMA,
            pltpu.SemaphoreType.DMA,
            pltpu.SemaphoreType.DMA,
            pltpu.SemaphoreType.DMA,
        ],
    )
    def gather_kernel(idx_hbm, table_hbm, out_hbm, idx_v, buf_a, buf_b,
                      gs_a, gs_b, os_a, os_b):
        wid = lax.axis_index("s") * NC + lax.axis_index("c")
        base = wid * bpw
        pltpu.sync_copy(idx_hbm.at[wid], idx_v)

        def start_gathers(s, buf, sem):
            for k in range(K):
                pltpu.async_copy(
                    table_hbm.at[idx_v.at[s * K + k]],
                    buf.at[pl.ds(k * CHUNK, CHUNK)], sem)

        def wait_gathers(s, buf, sem):
            for k in range(K):
                pltpu.make_async_copy(
                    table_hbm.at[idx_v.at[s * K + k]],
                    buf.at[pl.ds(k * CHUNK, CHUNK)], sem).wait()

        def out_copy(s, buf, sem):
            dst = out_hbm.at[pl.ds(base + s * SUPER, SUPER)]
            return pltpu.make_async_copy(buf, dst, sem)

        start_gathers(0, buf_a, gs_a)

        def body(it, _):
            s0 = it * 2
            s1 = s0 + 1
            wait_gathers(s0, buf_a, gs_a)

            @pl.when(it > 0)
            def _():
                out_copy(s1 - 2, buf_b, os_b).wait()

            start_gathers(s1, buf_b, gs_b)
            out_copy(s0, buf_a, os_a).start()
            wait_gathers(s1, buf_b, gs_b)
            out_copy(s0, buf_a, os_a).wait()

            @pl.when(s0 + 2 < nsuper)
            def _():
                start_gathers(s0 + 2, buf_a, gs_a)

            out_copy(s1, buf_b, os_b).start()
            return ()

        lax.fori_loop(0, nsuper // 2, body, (), unroll=False)
        out_copy(nsuper - 1, buf_b, os_b).wait()

    return gather_kernel


def kernel(multi_hot, table):
    rows, cols = multi_hot.shape
    B = rows * cols
    idx3 = multi_hot.astype(jnp.int32).reshape(NW, B // (NW * CHUNK), CHUNK)
    out = _make_gather(B)(idx3, table)
    return out.reshape(rows, cols, EMB)
